# baseline (speedup 1.0000x reference)
import functools

import jax
import jax.numpy as jnp
from jax import lax
from jax.experimental import pallas as pl
from jax.experimental.pallas import tpu as pltpu
from jax.experimental.pallas import tpu_sc as plsc

G = 48
NCELL = G * G * G
CIN = 128
COUT = 128

NC = 2
NS = 16
NW = NC * NS

NCHUNK = 12
CHUNK = NCELL // NCHUNK
SP_ROWS = CHUNK + 16
CH_PER_CORE = NCHUNK // NC
STRIPE = CHUNK // NS
PTS_I = NS * 26 * 128
TPTS = PTS_I // NS

RB = 4
FB = 64
TB = TPTS // FB

PTS_O = NW * 13 * 128
WPTS = PTS_O // NW
WG = WPTS // 128


def _cell16(pxb, pyb, pzb, s, iv):
    cx = (pxb[s] * iv).astype(jnp.int32)
    cy = (pyb[s] * iv).astype(jnp.int32)
    cz = (pzb[s] * iv).astype(jnp.int32)
    return cz * (G * G) + cy * G + cx


def _scatter_body(px_h, py_h, pz_h, feat_h, invv_h, grid_h,
                  pxb, pyb, pzb, ivb, linb, idxb, fb0, fb1, fb2, fb3,
                  sl0, sl1, sl2, sl3, sa0, sa1, sa2, sa3, spmem):
    core = lax.axis_index("c")
    sub = lax.axis_index("s")
    pbase = sub * TPTS
    fbuf = [fb0, fb1, fb2, fb3]
    semL = [sl0, sl1, sl2, sl3]
    semA = [sa0, sa1, sa2, sa3]

    def wait_load(r):
        pltpu.make_async_copy(feat_h.at[pl.ds(0, FB)], fbuf[r], semL[r]).wait()

    def wait_add(r):
        pltpu.make_async_copy(feat_h.at[pl.ds(0, FB)], fbuf[r], semA[r]).wait()

    pltpu.sync_copy(invv_h, ivb)
    pltpu.sync_copy(px_h.at[pl.ds(pbase, TPTS)], pxb)
    pltpu.sync_copy(py_h.at[pl.ds(pbase, TPTS)], pyb)
    pltpu.sync_copy(pz_h.at[pl.ds(pbase, TPTS)], pzb)
    iv = ivb[...]

    @pl.loop(0, TPTS // 16)
    def _(i):
        s = pl.ds(i * 16, 16)
        linb[s] = _cell16(pxb, pyb, pzb, s, iv)

    @pl.loop(0, CH_PER_CORE)
    def _(k):
        base = (core * CH_PER_CORE + k) * CHUNK

        @pl.loop(0, FB)
        def _(r):
            for j in range(8):
                fb0[r, pl.ds(j * 16, 16)] = jnp.zeros((16,), jnp.float32)

        for i in range(STRIPE // FB):
            pltpu.sync_copy(fb0, spmem.at[pl.ds(sub * STRIPE + i * FB, FB)])

        @pl.when(sub == 0)
        def _():
            pltpu.sync_copy(fb0.at[pl.ds(0, SP_ROWS - CHUNK)],
                            spmem.at[pl.ds(CHUNK, SP_ROWS - CHUNK)])

        plsc.subcore_barrier()

        @pl.loop(0, TB)
        def _(g):
            for j in range(FB // 16):
                s = pl.ds(g * FB + j * 16, 16)
                lin = linb[s]
                m = (lin >= base) & (lin < base + CHUNK)
                idxb[g, pl.ds(j * 16, 16)] = jnp.where(m, lin - base, CHUNK)

        for r in range(RB):
            pltpu.async_copy(feat_h.at[pl.ds(pbase + r * FB, FB)],
                             fbuf[r], semL[r])

        @pl.loop(0, TB // RB)
        def _(g):
            b0 = g * RB
            for r in range(RB):
                wait_load(r)
                pltpu.async_copy(fbuf[r], spmem.at[idxb.at[b0 + r]],
                                 semA[r], add=True)

            @pl.when(g < TB // RB - 1)
            def _():
                for r in range(RB):
                    wait_add(r)
                    pltpu.async_copy(
                        feat_h.at[pl.ds(pbase + (b0 + RB + r) * FB, FB)],
                        fbuf[r], semL[r])

        for r in range(RB):
            wait_add(r)

        plsc.subcore_barrier()

        for i in range(STRIPE // FB):
            r = i % 2
            if i >= 2:
                wait_add(r)
            pltpu.sync_copy(spmem.at[pl.ds(sub * STRIPE + i * FB, FB)], fbuf[r])
            pltpu.async_copy(fbuf[r],
                             grid_h.at[pl.ds(base + sub * STRIPE + i * FB, FB)],
                             semA[r])
        for r in range(2):
            wait_add(r)

        plsc.subcore_barrier()


def _gather_body(px_h, py_h, pz_h, invv_h, conv_h, out_h,
                 pxb, pyb, pzb, ivb, idxa, ob0, ob1, ob2, ob3,
                 sg0, sg1, sg2, sg3, sw0, sw1, sw2, sw3):
    core = lax.axis_index("c")
    sub = lax.axis_index("s")
    base = (sub * NC + core) * WPTS
    obuf = [ob0, ob1, ob2, ob3]
    semG = [sg0, sg1, sg2, sg3]
    semW = [sw0, sw1, sw2, sw3]

    def wait_g(r):
        pltpu.make_async_copy(conv_h.at[pl.ds(0, 128)], obuf[r], semG[r]).wait()

    def wait_w(r):
        pltpu.make_async_copy(conv_h.at[pl.ds(0, 128)], obuf[r], semW[r]).wait()

    pltpu.sync_copy(invv_h, ivb)
    pltpu.sync_copy(px_h.at[pl.ds(base, WPTS)], pxb)
    pltpu.sync_copy(py_h.at[pl.ds(base, WPTS)], pyb)
    pltpu.sync_copy(pz_h.at[pl.ds(base, WPTS)], pzb)
    iv = ivb[...]

    @pl.loop(0, WG)
    def _(g):
        for j in range(8):
            sl = pl.ds(g * 128 + j * 16, 16)
            idxa[g, pl.ds(j * 16, 16)] = _cell16(pxb, pyb, pzb, sl, iv)

    for g in range(WG):
        r = g % 4
        if g >= 4:
            wait_w(r)
        pltpu.async_copy(conv_h.at[idxa.at[g]], obuf[r], semG[r])
        if g >= 2:
            rp = (g - 2) % 4
            wait_g(rp)
            pltpu.async_copy(obuf[rp],
                             out_h.at[pl.ds(base + (g - 2) * 128, 128)],
                             semW[rp])
    for g in (WG - 2, WG - 1):
        rp = g % 4
        wait_g(rp)
        pltpu.async_copy(obuf[rp], out_h.at[pl.ds(base + g * 128, 128)],
                         semW[rp])
    for r in range(4):
        wait_w(r)


def _conv_body(gm1, g0, gp1, k9, bias, out, spad):
    z = pl.program_id(0)
    spad[...] = jnp.zeros((2448, 3 * CIN), jnp.bfloat16)

    @pl.when(z > 0)
    def _():
        spad[pl.ds(56, 2304), pl.ds(0, CIN)] = gm1[0].astype(jnp.bfloat16)

    spad[pl.ds(56, 2304), pl.ds(CIN, CIN)] = g0[0].astype(jnp.bfloat16)

    @pl.when(z < G - 1)
    def _():
        spad[pl.ds(56, 2304), pl.ds(2 * CIN, CIN)] = gp1[0].astype(jnp.bfloat16)

    xcol = lax.broadcasted_iota(jnp.int32, (2304, 1), 0) % G
    acc = jnp.zeros((2304, COUT), jnp.float32)
    for t in range(9):
        dy, dx = t // 3, t % 3
        off = (dy - 1) * G + (dx - 1)
        win = spad[pl.ds(56 + off, 2304), :]
        if dx != 1:
            xv = xcol + (dx - 1)
            win = jnp.where((xv >= 0) & (xv < G), win, jnp.bfloat16(0.0))
        acc = acc + jnp.dot(win, k9[t], preferred_element_type=jnp.float32)
    out[0] = acc + bias[...]


def _run_conv(grid3, k9, bias2, interpret=False):
    return pl.pallas_call(
        _conv_body,
        grid=(G,),
        in_specs=[
            pl.BlockSpec((1, G * G, CIN), lambda z: (jnp.maximum(z - 1, 0), 0, 0)),
            pl.BlockSpec((1, G * G, CIN), lambda z: (z, 0, 0)),
            pl.BlockSpec((1, G * G, CIN), lambda z: (jnp.minimum(z + 1, G - 1), 0, 0)),
            pl.BlockSpec((9, 3 * CIN, COUT), lambda z: (0, 0, 0)),
            pl.BlockSpec((1, COUT), lambda z: (0, 0)),
        ],
        out_specs=pl.BlockSpec((1, G * G, COUT), lambda z: (z, 0, 0)),
        out_shape=jax.ShapeDtypeStruct((G, G * G, COUT), jnp.float32),
        scratch_shapes=[pltpu.VMEM((2448, 3 * CIN), jnp.bfloat16)],
        interpret=interpret,
    )(grid3, grid3, grid3, k9, bias2)


def _run_scatter(px, py, pz, featp, invv):
    mesh = plsc.VectorSubcoreMesh(core_axis_name="c", subcore_axis_name="s",
                                  num_cores=NC, num_subcores=NS)
    f = functools.partial(
        pl.kernel,
        out_type=jax.ShapeDtypeStruct((NCELL, CIN), jnp.float32),
        mesh=mesh,
        scratch_types=[
            pltpu.VMEM((TPTS,), jnp.float32),
            pltpu.VMEM((TPTS,), jnp.float32),
            pltpu.VMEM((TPTS,), jnp.float32),
            pltpu.VMEM((16,), jnp.float32),
            pltpu.VMEM((TPTS,), jnp.int32),
            pltpu.VMEM((TB, FB), jnp.int32),
            pltpu.VMEM((FB, CIN), jnp.float32),
            pltpu.VMEM((FB, CIN), jnp.float32),
            pltpu.VMEM((FB, CIN), jnp.float32),
            pltpu.VMEM((FB, CIN), jnp.float32),
            pltpu.SemaphoreType.DMA,
            pltpu.SemaphoreType.DMA,
            pltpu.SemaphoreType.DMA,
            pltpu.SemaphoreType.DMA,
            pltpu.SemaphoreType.DMA,
            pltpu.SemaphoreType.DMA,
            pltpu.SemaphoreType.DMA,
            pltpu.SemaphoreType.DMA,
            pltpu.VMEM_SHARED((SP_ROWS, CIN), jnp.float32),
        ],
    )(_scatter_body)
    return f(px, py, pz, featp, invv)


def _run_gather(px, py, pz, invv, convf):
    mesh = plsc.VectorSubcoreMesh(core_axis_name="c", subcore_axis_name="s",
                                  num_cores=NC, num_subcores=NS)
    f = functools.partial(
        pl.kernel,
        out_type=jax.ShapeDtypeStruct((PTS_O, COUT), jnp.float32),
        mesh=mesh,
        scratch_types=[
            pltpu.VMEM((WPTS,), jnp.float32),
            pltpu.VMEM((WPTS,), jnp.float32),
            pltpu.VMEM((WPTS,), jnp.float32),
            pltpu.VMEM((16,), jnp.float32),
            pltpu.VMEM((WG, 128), jnp.int32),
            pltpu.VMEM((128, COUT), jnp.float32),
            pltpu.VMEM((128, COUT), jnp.float32),
            pltpu.VMEM((128, COUT), jnp.float32),
            pltpu.VMEM((128, COUT), jnp.float32),
            pltpu.SemaphoreType.DMA,
            pltpu.SemaphoreType.DMA,
            pltpu.SemaphoreType.DMA,
            pltpu.SemaphoreType.DMA,
            pltpu.SemaphoreType.DMA,
            pltpu.SemaphoreType.DMA,
            pltpu.SemaphoreType.DMA,
            pltpu.SemaphoreType.DMA,
        ],
    )(_gather_body)
    return f(px, py, pz, invv, convf)


def kernel(inp_features, inp_positions, out_positions, kernel, bias, voxel_size):
    n_in = inp_positions.shape[0]
    n_out = out_positions.shape[0]
    f32 = jnp.float32
    bf16 = jnp.bfloat16

    invv = jnp.full((16,), 1.0, f32) / jnp.asarray(voxel_size, f32)

    pad_i = PTS_I - n_in
    ppos = inp_positions.astype(f32)
    px = jnp.concatenate([ppos[:, 0], jnp.full((pad_i,), 100.5, f32)])
    py = jnp.concatenate([ppos[:, 1], jnp.full((pad_i,), 100.5, f32)])
    pz = jnp.concatenate([ppos[:, 2], jnp.full((pad_i,), 100.5, f32)])
    featp = jnp.concatenate([inp_features.astype(f32),
                             jnp.zeros((pad_i, CIN), f32)])

    grid = _run_scatter(px, py, pz, featp, invv)

    k9 = kernel.astype(bf16).transpose(1, 2, 0, 3, 4).reshape(9, 3 * CIN, COUT)
    conv = _run_conv(grid.reshape(G, G * G, CIN), k9,
                     bias.astype(f32).reshape(1, COUT))
    convf = conv.reshape(NCELL, COUT)

    pad_o = PTS_O - n_out
    qpos = out_positions.astype(f32)
    qx = jnp.concatenate([qpos[:, 0], jnp.full((pad_o,), 0.5, f32)])
    qy = jnp.concatenate([qpos[:, 1], jnp.full((pad_o,), 0.5, f32)])
    qz = jnp.concatenate([qpos[:, 2], jnp.full((pad_o,), 0.5, f32)])

    outp = _run_gather(qx, qy, qz, invv, convf)
    return outp[:n_out]


# speedup vs baseline: 1.0986x; 1.0986x over previous
import functools

import jax
import jax.numpy as jnp
from jax import lax
from jax.experimental import pallas as pl
from jax.experimental.pallas import tpu as pltpu
from jax.experimental.pallas import tpu_sc as plsc

G = 48
NCELL = G * G * G
CIN = 128
COUT = 128

NC = 2
NS = 16
NW = NC * NS

NCHUNK = 12
CHUNK = NCELL // NCHUNK
SP_ROWS = CHUNK + 16
CH_PER_CORE = NCHUNK // NC
STRIPE = CHUNK // NS
PTS_I = NS * 26 * 128
TPTS = PTS_I // NS

RB = 4
FB = 64
TB = TPTS // FB

PTS_O = NW * 13 * 128
WPTS = PTS_O // NW
WG = WPTS // 128

GW = G * 50
WPAD = 2504


def _cell16(pxb, pyb, pzb, s, iv):
    cx = (pxb[s] * iv).astype(jnp.int32)
    cy = (pyb[s] * iv).astype(jnp.int32)
    cz = (pzb[s] * iv).astype(jnp.int32)
    return cz * (G * G) + cy * G + cx


def _scatter_body(px_h, py_h, pz_h, feat_h, invv_h, grid_h,
                  pxb, pyb, pzb, ivb, linb, idxb, fb0, fb1, fb2, fb3,
                  sl0, sl1, sl2, sl3, sa0, sa1, sa2, sa3, spmem):
    core = lax.axis_index("c")
    sub = lax.axis_index("s")
    pbase = sub * TPTS
    fbuf = [fb0, fb1, fb2, fb3]
    semL = [sl0, sl1, sl2, sl3]
    semA = [sa0, sa1, sa2, sa3]

    def wait_load(r):
        pltpu.make_async_copy(feat_h.at[pl.ds(0, FB)], fbuf[r], semL[r]).wait()

    def wait_add(r):
        pltpu.make_async_copy(feat_h.at[pl.ds(0, FB)], fbuf[r], semA[r]).wait()

    pltpu.sync_copy(invv_h, ivb)
    pltpu.sync_copy(px_h.at[pl.ds(pbase, TPTS)], pxb)
    pltpu.sync_copy(py_h.at[pl.ds(pbase, TPTS)], pyb)
    pltpu.sync_copy(pz_h.at[pl.ds(pbase, TPTS)], pzb)
    iv = ivb[...]

    @pl.loop(0, TPTS // 16)
    def _(i):
        s = pl.ds(i * 16, 16)
        linb[s] = _cell16(pxb, pyb, pzb, s, iv)

    @pl.loop(0, CH_PER_CORE)
    def _(k):
        base = (core * CH_PER_CORE + k) * CHUNK

        @pl.loop(0, FB)
        def _(r):
            for j in range(8):
                fb0[r, pl.ds(j * 16, 16)] = jnp.zeros((16,), jnp.float32)

        for i in range(STRIPE // FB):
            pltpu.sync_copy(fb0, spmem.at[pl.ds(sub * STRIPE + i * FB, FB)])

        @pl.when(sub == 0)
        def _():
            pltpu.sync_copy(fb0.at[pl.ds(0, SP_ROWS - CHUNK)],
                            spmem.at[pl.ds(CHUNK, SP_ROWS - CHUNK)])

        plsc.subcore_barrier()

        @pl.loop(0, TB)
        def _(g):
            for j in range(FB // 16):
                s = pl.ds(g * FB + j * 16, 16)
                lin = linb[s]
                m = (lin >= base) & (lin < base + CHUNK)
                idxb[g, pl.ds(j * 16, 16)] = jnp.where(m, lin - base, CHUNK)

        for r in range(RB):
            pltpu.async_copy(feat_h.at[pl.ds(pbase + r * FB, FB)],
                             fbuf[r], semL[r])

        @pl.loop(0, TB // RB)
        def _(g):
            b0 = g * RB
            for r in range(RB):
                wait_load(r)
                pltpu.async_copy(fbuf[r], spmem.at[idxb.at[b0 + r]],
                                 semA[r], add=True)

            @pl.when(g < TB // RB - 1)
            def _():
                for r in range(RB):
                    wait_add(r)
                    pltpu.async_copy(
                        feat_h.at[pl.ds(pbase + (b0 + RB + r) * FB, FB)],
                        fbuf[r], semL[r])

        for r in range(RB):
            wait_add(r)

        plsc.subcore_barrier()

        for i in range(STRIPE // FB):
            r = i % 2
            if i >= 2:
                wait_add(r)
            pltpu.sync_copy(spmem.at[pl.ds(sub * STRIPE + i * FB, FB)], fbuf[r])
            pltpu.async_copy(fbuf[r],
                             grid_h.at[pl.ds(base + sub * STRIPE + i * FB, FB)],
                             semA[r])
        for r in range(2):
            wait_add(r)

        plsc.subcore_barrier()


def _gather_body(px_h, py_h, pz_h, invv_h, conv_h, out_h,
                 pxb, pyb, pzb, ivb, idxa, ob0, ob1, ob2, ob3,
                 sg0, sg1, sg2, sg3, sw0, sw1, sw2, sw3):
    core = lax.axis_index("c")
    sub = lax.axis_index("s")
    base = (sub * NC + core) * WPTS
    obuf = [ob0, ob1, ob2, ob3]
    semG = [sg0, sg1, sg2, sg3]
    semW = [sw0, sw1, sw2, sw3]

    def wait_g(r):
        pltpu.make_async_copy(conv_h.at[pl.ds(0, 128)], obuf[r], semG[r]).wait()

    def wait_w(r):
        pltpu.make_async_copy(conv_h.at[pl.ds(0, 128)], obuf[r], semW[r]).wait()

    pltpu.sync_copy(invv_h, ivb)
    pltpu.sync_copy(px_h.at[pl.ds(base, WPTS)], pxb)
    pltpu.sync_copy(py_h.at[pl.ds(base, WPTS)], pyb)
    pltpu.sync_copy(pz_h.at[pl.ds(base, WPTS)], pzb)
    iv = ivb[...]

    @pl.loop(0, WG)
    def _(g):
        for j in range(8):
            sl = pl.ds(g * 128 + j * 16, 16)
            cx = (pxb[sl] * iv).astype(jnp.int32)
            cy = (pyb[sl] * iv).astype(jnp.int32)
            cz = (pzb[sl] * iv).astype(jnp.int32)
            idxa[g, pl.ds(j * 16, 16)] = cz * GW + cy * 50 + cx

    for g in range(WG):
        r = g % 4
        if g >= 4:
            wait_w(r)
        pltpu.async_copy(conv_h.at[idxa.at[g]], obuf[r], semG[r])
        if g >= 2:
            rp = (g - 2) % 4
            wait_g(rp)
            pltpu.async_copy(obuf[rp],
                             out_h.at[pl.ds(base + (g - 2) * 128, 128)],
                             semW[rp])
    for g in (WG - 2, WG - 1):
        rp = g % 4
        wait_g(rp)
        pltpu.async_copy(obuf[rp], out_h.at[pl.ds(base + g * 128, 128)],
                         semW[rp])
    for r in range(4):
        wait_w(r)


def _conv_body(gm1, g0, gp1, k9, bias, out, spad):
    z = pl.program_id(0)

    @pl.when(z == 0)
    def _():
        spad[...] = jnp.zeros((WPAD, 3 * CIN), jnp.float32)

    for s, (ref, lo, hi) in enumerate(((gm1, 1, None), (g0, None, None),
                                       (gp1, None, G - 1))):
        valid = True
        if lo is not None:
            valid = z >= lo
        if hi is not None:
            valid = z < hi
        cb = pl.ds(s * CIN, CIN)

        @pl.when(valid)
        def _():
            for y in range(G):
                spad[pl.ds(y * 50 + 51, G), cb] = ref[0, pl.ds(y * G, G), :]

        @pl.when(jnp.logical_not(valid))
        def _():
            for y in range(G):
                spad[pl.ds(y * 50 + 51, G), cb] = jnp.zeros((G, CIN),
                                                            jnp.float32)

    acc = jnp.zeros((GW, COUT), jnp.float32)
    for t in range(9):
        dy, dx = t // 3, t % 3
        off = (dy - 1) * 50 + (dx - 1)
        win = spad[pl.ds(51 + off, GW), :]
        acc = acc + jnp.dot(win, k9[t], preferred_element_type=jnp.float32)
    out[0] = acc + bias[...]


def _run_conv(grid3, k9, bias2, interpret=False):
    return pl.pallas_call(
        _conv_body,
        grid=(G,),
        in_specs=[
            pl.BlockSpec((1, G * G, CIN), lambda z: (jnp.maximum(z - 1, 0), 0, 0)),
            pl.BlockSpec((1, G * G, CIN), lambda z: (z, 0, 0)),
            pl.BlockSpec((1, G * G, CIN), lambda z: (jnp.minimum(z + 1, G - 1), 0, 0)),
            pl.BlockSpec((9, 3 * CIN, COUT), lambda z: (0, 0, 0)),
            pl.BlockSpec((1, COUT), lambda z: (0, 0)),
        ],
        out_specs=pl.BlockSpec((1, GW, COUT), lambda z: (z, 0, 0)),
        out_shape=jax.ShapeDtypeStruct((G, GW, COUT), jnp.float32),
        scratch_shapes=[pltpu.VMEM((WPAD, 3 * CIN), jnp.float32)],
        interpret=interpret,
    )(grid3, grid3, grid3, k9, bias2)


def _run_scatter(px, py, pz, featp, invv):
    mesh = plsc.VectorSubcoreMesh(core_axis_name="c", subcore_axis_name="s",
                                  num_cores=NC, num_subcores=NS)
    f = functools.partial(
        pl.kernel,
        out_type=jax.ShapeDtypeStruct((NCELL, CIN), jnp.float32),
        mesh=mesh,
        scratch_types=[
            pltpu.VMEM((TPTS,), jnp.float32),
            pltpu.VMEM((TPTS,), jnp.float32),
            pltpu.VMEM((TPTS,), jnp.float32),
            pltpu.VMEM((16,), jnp.float32),
            pltpu.VMEM((TPTS,), jnp.int32),
            pltpu.VMEM((TB, FB), jnp.int32),
            pltpu.VMEM((FB, CIN), jnp.float32),
            pltpu.VMEM((FB, CIN), jnp.float32),
            pltpu.VMEM((FB, CIN), jnp.float32),
            pltpu.VMEM((FB, CIN), jnp.float32),
            pltpu.SemaphoreType.DMA,
            pltpu.SemaphoreType.DMA,
            pltpu.SemaphoreType.DMA,
            pltpu.SemaphoreType.DMA,
            pltpu.SemaphoreType.DMA,
            pltpu.SemaphoreType.DMA,
            pltpu.SemaphoreType.DMA,
            pltpu.SemaphoreType.DMA,
            pltpu.VMEM_SHARED((SP_ROWS, CIN), jnp.float32),
        ],
    )(_scatter_body)
    return f(px, py, pz, featp, invv)


def _run_gather(px, py, pz, invv, convf):
    mesh = plsc.VectorSubcoreMesh(core_axis_name="c", subcore_axis_name="s",
                                  num_cores=NC, num_subcores=NS)
    f = functools.partial(
        pl.kernel,
        out_type=jax.ShapeDtypeStruct((PTS_O, COUT), jnp.float32),
        mesh=mesh,
        scratch_types=[
            pltpu.VMEM((WPTS,), jnp.float32),
            pltpu.VMEM((WPTS,), jnp.float32),
            pltpu.VMEM((WPTS,), jnp.float32),
            pltpu.VMEM((16,), jnp.float32),
            pltpu.VMEM((WG, 128), jnp.int32),
            pltpu.VMEM((128, COUT), jnp.float32),
            pltpu.VMEM((128, COUT), jnp.float32),
            pltpu.VMEM((128, COUT), jnp.float32),
            pltpu.VMEM((128, COUT), jnp.float32),
            pltpu.SemaphoreType.DMA,
            pltpu.SemaphoreType.DMA,
            pltpu.SemaphoreType.DMA,
            pltpu.SemaphoreType.DMA,
            pltpu.SemaphoreType.DMA,
            pltpu.SemaphoreType.DMA,
            pltpu.SemaphoreType.DMA,
            pltpu.SemaphoreType.DMA,
        ],
    )(_gather_body)
    return f(px, py, pz, invv, convf)


def kernel(inp_features, inp_positions, out_positions, kernel, bias, voxel_size):
    n_in = inp_positions.shape[0]
    n_out = out_positions.shape[0]
    f32 = jnp.float32
    bf16 = jnp.bfloat16

    invv = jnp.full((16,), 1.0, f32) / jnp.asarray(voxel_size, f32)

    pad_i = PTS_I - n_in
    ppos = inp_positions.astype(f32)
    px = jnp.concatenate([ppos[:, 0], jnp.full((pad_i,), 100.5, f32)])
    py = jnp.concatenate([ppos[:, 1], jnp.full((pad_i,), 100.5, f32)])
    pz = jnp.concatenate([ppos[:, 2], jnp.full((pad_i,), 100.5, f32)])
    featp = jnp.concatenate([inp_features.astype(f32),
                             jnp.zeros((pad_i, CIN), f32)])

    grid = _run_scatter(px, py, pz, featp, invv)

    k9 = kernel.astype(f32).transpose(1, 2, 0, 3, 4).reshape(9, 3 * CIN, COUT)
    conv = _run_conv(grid.reshape(G, G * G, CIN), k9,
                     bias.astype(f32).reshape(1, COUT))
    convf = conv.reshape(G * GW, COUT)

    pad_o = PTS_O - n_out
    qpos = out_positions.astype(f32)
    qx = jnp.concatenate([qpos[:, 0], jnp.full((pad_o,), 0.5, f32)])
    qy = jnp.concatenate([qpos[:, 1], jnp.full((pad_o,), 0.5, f32)])
    qz = jnp.concatenate([qpos[:, 2], jnp.full((pad_o,), 0.5, f32)])

    outp = _run_gather(qx, qy, qz, invv, convf)
    return outp[:n_out]
